# Initial kernel scaffold; baseline (speedup 1.0000x reference)
import jax
import jax.numpy as jnp
from jax.experimental import pallas as pl


def kernel(instruments, style, tempo, pitch, duration, instrument_table, style_table, W_tempo, b_tempo, W_pitch, b_pitch, W_dur, b_dur, W_proj, b_proj):
    raise NotImplementedError("write your pallas kernel here")


# speedup vs baseline: 13.1139x; 13.1139x over previous
import jax
import jax.numpy as jnp
from jax.experimental import pallas as pl
from jax.experimental.pallas import tpu as pltpu

_K = 256
_STYLE_OFF = 128
_TPD_OFF = 192


def _prep_kernel(inst_pad_ref, wpi_t_ref, style_pad_ref, wps_t_ref,
                 v_ref, wptpd_t_ref, b_stack_ref, b_proj_ref,
                 a_ref, beff_ref):
    a_ref[0:128, :] = jnp.dot(inst_pad_ref[...], wpi_t_ref[...],
                              preferred_element_type=jnp.float32)
    a_ref[128:192, :] = jnp.dot(style_pad_ref[...], wps_t_ref[...],
                                preferred_element_type=jnp.float32)
    a_ref[192:200, :] = jnp.dot(v_ref[...], wptpd_t_ref[...],
                                preferred_element_type=jnp.float32)
    a_ref[200:256, :] = jnp.zeros((56, a_ref.shape[1]), jnp.float32)
    beff_ref[...] = (jnp.dot(b_stack_ref[...], wptpd_t_ref[...],
                             preferred_element_type=jnp.float32)
                     + b_proj_ref[...])


def _main_kernel(inst_ref, sty_ref, tempo_ref, pitch_ref, dur_ref,
                 a_ref, beff_ref, out_ref):
    idx = inst_ref[...]
    bsz, L = idx.shape
    iota = jax.lax.broadcasted_iota(jnp.int32, (bsz, _K), 1)
    m = ((sty_ref[...] + _STYLE_OFF) == iota).astype(jnp.float32)
    for l in range(L):
        m = m + (idx[:, l:l + 1] == iota).astype(jnp.float32)
    m = jnp.where(iota == _TPD_OFF, tempo_ref[...], m)
    m = jnp.where(iota == _TPD_OFF + 1, pitch_ref[...], m)
    m = jnp.where(iota == _TPD_OFF + 2, dur_ref[...], m)
    out_ref[...] = (jnp.dot(m, a_ref[...], preferred_element_type=jnp.float32)
                    + beff_ref[...])


def kernel(instruments, style, tempo, pitch, duration,
           instrument_table, style_table,
           W_tempo, b_tempo, W_pitch, b_pitch, W_dur, b_dur,
           W_proj, b_proj):
    B, L = instruments.shape
    n_inst, d_inst = instrument_table.shape
    n_style, d_style = style_table.shape
    H = W_proj.shape[0]

    f32 = jnp.float32
    inst_pad = jnp.zeros((128, d_inst), f32).at[:n_inst].set(instrument_table)
    style_pad = jnp.zeros((64, d_style), f32).at[:n_style].set(style_table)
    wpi_t = W_proj[:, :d_inst].T
    wps_t = W_proj[:, d_inst:d_inst + d_style].T
    wptpd_t = W_proj[:, d_inst + d_style:].T
    v = jnp.zeros((8, 192), f32)
    v = v.at[0, 0:64].set(W_tempo[:, 0])
    v = v.at[1, 64:128].set(W_pitch[:, 0])
    v = v.at[2, 128:192].set(W_dur[:, 0])
    b_stack = jnp.concatenate([b_tempo, b_pitch, b_dur]).reshape(1, 192)
    b_proj2 = b_proj.reshape(1, H).astype(f32)

    a, beff = pl.pallas_call(
        _prep_kernel,
        out_shape=(jax.ShapeDtypeStruct((_K, H), f32),
                   jax.ShapeDtypeStruct((1, H), f32)),
    )(inst_pad, wpi_t, style_pad, wps_t, v, wptpd_t, b_stack, b_proj2)

    sty2 = style.reshape(B, 1).astype(jnp.int32)
    bsz = 512
    grid = (B // bsz,)
    out = pl.pallas_call(
        _main_kernel,
        grid=grid,
        in_specs=[
            pl.BlockSpec((bsz, L), lambda i: (i, 0)),
            pl.BlockSpec((bsz, 1), lambda i: (i, 0)),
            pl.BlockSpec((bsz, 1), lambda i: (i, 0)),
            pl.BlockSpec((bsz, 1), lambda i: (i, 0)),
            pl.BlockSpec((bsz, 1), lambda i: (i, 0)),
            pl.BlockSpec((_K, H), lambda i: (0, 0)),
            pl.BlockSpec((1, H), lambda i: (0, 0)),
        ],
        out_specs=pl.BlockSpec((bsz, H), lambda i: (i, 0)),
        out_shape=jax.ShapeDtypeStruct((B, H), f32),
        compiler_params=pltpu.CompilerParams(
            dimension_semantics=("parallel",)),
    )(instruments.astype(jnp.int32), sty2, tempo, pitch, duration, a, beff)
    return out


# speedup vs baseline: 13.4530x; 1.0259x over previous
import jax
import jax.numpy as jnp
from jax.experimental import pallas as pl
from jax.experimental.pallas import tpu as pltpu

_K = 256
_STYLE_OFF = 128
_TPD_OFF = 192


def _prep_kernel(inst_pad_ref, wpi_t_ref, style_pad_ref, wps_t_ref,
                 v_ref, wptpd_t_ref, b_stack_ref, b_proj_ref,
                 a_ref, beff_ref):
    a_ref[0:128, :] = jnp.dot(inst_pad_ref[...], wpi_t_ref[...],
                              preferred_element_type=jnp.float32)
    a_ref[128:192, :] = jnp.dot(style_pad_ref[...], wps_t_ref[...],
                                preferred_element_type=jnp.float32)
    a_ref[192:200, :] = jnp.dot(v_ref[...], wptpd_t_ref[...],
                                preferred_element_type=jnp.float32)
    a_ref[200:256, :] = jnp.zeros((56, a_ref.shape[1]), jnp.float32)
    beff_ref[...] = (jnp.dot(b_stack_ref[...], wptpd_t_ref[...],
                             preferred_element_type=jnp.float32)
                     + b_proj_ref[...])


def _main_kernel(inst_ref, sty_ref, tempo_ref, pitch_ref, dur_ref,
                 a_ref, beff_ref, out_ref):
    idx = inst_ref[...]
    bsz, L = idx.shape
    iota_i = jax.lax.broadcasted_iota(jnp.int32, (bsz, 128), 1)
    mi = jnp.zeros((bsz, 128), jnp.float32)
    for l in range(L):
        mi = mi + (idx[:, l:l + 1] == iota_i).astype(jnp.float32)
    iota_s = jax.lax.broadcasted_iota(jnp.int32, (bsz, 64), 1)
    ms = (sty_ref[...] == iota_s).astype(jnp.float32)
    mt = jnp.where(iota_s == 0, tempo_ref[...],
                   jnp.where(iota_s == 1, pitch_ref[...],
                             jnp.where(iota_s == 2, dur_ref[...], 0.0)))
    m = jnp.concatenate([mi, ms, mt], axis=1)
    out_ref[...] = (jnp.dot(m, a_ref[...], preferred_element_type=jnp.float32)
                    + beff_ref[...])


def kernel(instruments, style, tempo, pitch, duration,
           instrument_table, style_table,
           W_tempo, b_tempo, W_pitch, b_pitch, W_dur, b_dur,
           W_proj, b_proj):
    B, L = instruments.shape
    n_inst, d_inst = instrument_table.shape
    n_style, d_style = style_table.shape
    H = W_proj.shape[0]

    f32 = jnp.float32
    inst_pad = jnp.zeros((128, d_inst), f32).at[:n_inst].set(instrument_table)
    style_pad = jnp.zeros((64, d_style), f32).at[:n_style].set(style_table)
    wpi_t = W_proj[:, :d_inst].T
    wps_t = W_proj[:, d_inst:d_inst + d_style].T
    wptpd_t = W_proj[:, d_inst + d_style:].T
    v = jnp.zeros((8, 192), f32)
    v = v.at[0, 0:64].set(W_tempo[:, 0])
    v = v.at[1, 64:128].set(W_pitch[:, 0])
    v = v.at[2, 128:192].set(W_dur[:, 0])
    b_stack = jnp.concatenate([b_tempo, b_pitch, b_dur]).reshape(1, 192)
    b_proj2 = b_proj.reshape(1, H).astype(f32)

    a, beff = pl.pallas_call(
        _prep_kernel,
        out_shape=(jax.ShapeDtypeStruct((_K, H), f32),
                   jax.ShapeDtypeStruct((1, H), f32)),
    )(inst_pad, wpi_t, style_pad, wps_t, v, wptpd_t, b_stack, b_proj2)

    sty2 = style.reshape(B, 1).astype(jnp.int32)
    bsz = 512
    grid = (B // bsz,)
    out = pl.pallas_call(
        _main_kernel,
        grid=grid,
        in_specs=[
            pl.BlockSpec((bsz, L), lambda i: (i, 0)),
            pl.BlockSpec((bsz, 1), lambda i: (i, 0)),
            pl.BlockSpec((bsz, 1), lambda i: (i, 0)),
            pl.BlockSpec((bsz, 1), lambda i: (i, 0)),
            pl.BlockSpec((bsz, 1), lambda i: (i, 0)),
            pl.BlockSpec((_K, H), lambda i: (0, 0)),
            pl.BlockSpec((1, H), lambda i: (0, 0)),
        ],
        out_specs=pl.BlockSpec((bsz, H), lambda i: (i, 0)),
        out_shape=jax.ShapeDtypeStruct((B, H), f32),
        compiler_params=pltpu.CompilerParams(
            dimension_semantics=("parallel",)),
    )(instruments.astype(jnp.int32), sty2, tempo, pitch, duration, a, beff)
    return out


# speedup vs baseline: 30.0745x; 2.2355x over previous
import jax
import jax.numpy as jnp
from jax.experimental import pallas as pl
from jax.experimental.pallas import tpu as pltpu

_K = 256
_STYLE_OFF = 128
_TPD_OFF = 192


def _prep_kernel(inst_pad_ref, wpi_t_ref, style_pad_ref, wps_t_ref,
                 v_ref, wptpd_t_ref, b_stack_ref, b_proj_ref,
                 a_ref, beff_ref):
    a_ref[0:128, :] = jnp.dot(inst_pad_ref[...], wpi_t_ref[...],
                              preferred_element_type=jnp.float32)
    a_ref[128:192, :] = jnp.dot(style_pad_ref[...], wps_t_ref[...],
                                preferred_element_type=jnp.float32)
    a_ref[192:200, :] = jnp.dot(v_ref[...], wptpd_t_ref[...],
                                preferred_element_type=jnp.float32)
    a_ref[200:256, :] = jnp.zeros((56, a_ref.shape[1]), jnp.float32)
    beff_ref[...] = (jnp.dot(b_stack_ref[...], wptpd_t_ref[...],
                             preferred_element_type=jnp.float32)
                     + b_proj_ref[...])


def _main_kernel(inst_ref, sty_ref, tempo_ref, pitch_ref, dur_ref,
                 a_ref, beff_ref, out_ref):
    idx = inst_ref[...].astype(jnp.bfloat16)
    L, bsz = idx.shape
    iota_i = jax.lax.broadcasted_iota(jnp.int32, (128, bsz), 0).astype(
        jnp.bfloat16)
    mi = jnp.zeros((128, bsz), jnp.bfloat16)
    one = jnp.ones((), jnp.bfloat16)
    for l in range(L):
        mi = jnp.where(idx[l:l + 1, :] == iota_i, mi + one, mi)
    iota_s = jax.lax.broadcasted_iota(jnp.int32, (64, bsz), 0).astype(
        jnp.bfloat16)
    styb = sty_ref[...].astype(jnp.bfloat16)
    ms = (styb == iota_s).astype(jnp.bfloat16)
    mt = jnp.where(iota_s == 0, tempo_ref[...].astype(jnp.bfloat16),
                   jnp.where(iota_s == 1, pitch_ref[...].astype(jnp.bfloat16),
                             jnp.where(iota_s == 2,
                                       dur_ref[...].astype(jnp.bfloat16),
                                       jnp.zeros((), jnp.bfloat16))))
    m = jnp.concatenate([mi, ms, mt], axis=0)
    out_ref[...] = jax.lax.dot_general(
        m, a_ref[...].astype(jnp.bfloat16),
        dimension_numbers=(((0,), (0,)), ((), ())),
        preferred_element_type=jnp.float32) + beff_ref[...]


def kernel(instruments, style, tempo, pitch, duration,
           instrument_table, style_table,
           W_tempo, b_tempo, W_pitch, b_pitch, W_dur, b_dur,
           W_proj, b_proj):
    B, L = instruments.shape
    n_inst, d_inst = instrument_table.shape
    n_style, d_style = style_table.shape
    H = W_proj.shape[0]

    f32 = jnp.float32
    inst_pad = jnp.zeros((128, d_inst), f32).at[:n_inst].set(instrument_table)
    style_pad = jnp.zeros((64, d_style), f32).at[:n_style].set(style_table)
    wpi_t = W_proj[:, :d_inst].T
    wps_t = W_proj[:, d_inst:d_inst + d_style].T
    wptpd_t = W_proj[:, d_inst + d_style:].T
    v = jnp.zeros((8, 192), f32)
    v = v.at[0, 0:64].set(W_tempo[:, 0])
    v = v.at[1, 64:128].set(W_pitch[:, 0])
    v = v.at[2, 128:192].set(W_dur[:, 0])
    b_stack = jnp.concatenate([b_tempo, b_pitch, b_dur]).reshape(1, 192)
    b_proj2 = b_proj.reshape(1, H).astype(f32)

    a, beff = pl.pallas_call(
        _prep_kernel,
        out_shape=(jax.ShapeDtypeStruct((_K, H), f32),
                   jax.ShapeDtypeStruct((1, H), f32)),
    )(inst_pad, wpi_t, style_pad, wps_t, v, wptpd_t, b_stack, b_proj2)

    inst_t = instruments.astype(jnp.int32).T
    sty_t = style.reshape(1, B).astype(jnp.int32)
    tempo_t = tempo.reshape(1, B)
    pitch_t = pitch.reshape(1, B)
    dur_t = duration.reshape(1, B)
    bsz = 512
    grid = (B // bsz,)
    out = pl.pallas_call(
        _main_kernel,
        grid=grid,
        in_specs=[
            pl.BlockSpec((L, bsz), lambda i: (0, i)),
            pl.BlockSpec((1, bsz), lambda i: (0, i)),
            pl.BlockSpec((1, bsz), lambda i: (0, i)),
            pl.BlockSpec((1, bsz), lambda i: (0, i)),
            pl.BlockSpec((1, bsz), lambda i: (0, i)),
            pl.BlockSpec((_K, H), lambda i: (0, 0)),
            pl.BlockSpec((1, H), lambda i: (0, 0)),
        ],
        out_specs=pl.BlockSpec((bsz, H), lambda i: (i, 0)),
        out_shape=jax.ShapeDtypeStruct((B, H), f32),
        compiler_params=pltpu.CompilerParams(
            dimension_semantics=("parallel",)),
    )(inst_t, sty_t, tempo_t, pitch_t, dur_t, a, beff)
    return out
